# baseline (speedup 1.0000x reference)
import functools

import jax
import jax.numpy as jnp
from jax import lax
from jax.experimental import pallas as pl
from jax.experimental.pallas import tpu as pltpu
from jax.experimental.pallas import tpu_sc as plsc

VOCAB = 1000000
EMBED_DIM = 64
BATCH = 16384

_NC = 2
_NS = 16
_NW = _NC * _NS
_BPW = BATCH // _NW
_LANES = 16

_TAIL_BASE = (VOCAB // 128) * 128
_NPANEL = _TAIL_BASE // 128
_PPT = 244
_XBASE = _PPT * _NW
_WIN = 2
_NWIN = _PPT // _WIN
_RING = 8
_PP_EXTRA = 252
_PP_TAIL = 254
_NHIT = 32768
_STAGE = 64
_TRASH = 2 * BATCH
_EROWS = 2 * BATCH + _STAGE
_SCAN = 2048


def _k1_extract(target, context, table_t, tail_t):
    mesh = plsc.VectorSubcoreMesh(core_axis_name="c", subcore_axis_name="s")

    @functools.partial(
        pl.kernel,
        mesh=mesh,
        out_type=jax.ShapeDtypeStruct((_EROWS, 128), jnp.float32),
        compiler_params=pltpu.CompilerParams(needs_layout_passes=False),
        scratch_types=[
            pltpu.VMEM((_SCAN,), jnp.int32),
            pltpu.VMEM((_NHIT,), jnp.int32),
            pltpu.VMEM((_RING * EMBED_DIM, 128), jnp.float32),
            pltpu.VMEM((_STAGE, 128), jnp.float32),
            pltpu.VMEM((_STAGE,), jnp.int32),
            pltpu.VMEM((EMBED_DIM, EMBED_DIM), jnp.float32),
            pltpu.VMEM((_LANES, 128), jnp.int32),
            pltpu.VMEM((_LANES, 128), jnp.int32),
            pltpu.VMEM((8, _LANES), jnp.int32),
            pltpu.VMEM((8, _LANES), jnp.int32),
            pltpu.SemaphoreType.DMA,
            pltpu.SemaphoreType.DMA,
        ],
    )
    def k1(tgt_hbm, ctx_hbm, table_hbm, tail_hbm, embeds_hbm,
           scanbuf, hits2, ring, stage, stagepos, tailbuf,
           counts2d, cursor2d, offw2d, ends2d, sem0, sem1):
        wid = lax.axis_index("s") * _NC + lax.axis_index("c")
        lo = wid * _PPT
        xpanel = _XBASE + wid
        has_extra = wid < 4
        is_last = wid == _NW - 1
        lane = lax.iota(jnp.int32, _LANES)

        pltpu.sync_copy(tail_hbm, tailbuf)

        for l in range(_LANES):
            for cc in range(8):
                counts2d[l, pl.ds(cc * _LANES, _LANES)] = jnp.zeros(
                    (_LANES,), jnp.int32)

        def scan_array(arr_hbm, pos_off, histogram):
            def chunk(cb, _):
                pltpu.sync_copy(arr_hbm.at[pl.ds(cb * _SCAN, _SCAN)], scanbuf)

                def vreg(k, _):
                    v = scanbuf[pl.ds(k * _LANES, _LANES)]
                    p = lax.shift_right_logical(v, 7)
                    in_main = (p >= lo) & (p < lo + _PPT)
                    is_x = has_extra & (p == xpanel)
                    is_t = is_last & (p == _NPANEL)
                    m = in_main | is_x | is_t
                    mi = m.astype(jnp.int32)
                    pp = jnp.where(is_x, _PP_EXTRA,
                                   jnp.where(is_t, _PP_TAIL, (p - lo) & 255))
                    w = jnp.where(m, lax.shift_right_logical(pp, 1), 0)
                    if histogram:
                        plsc.addupdate_scatter(counts2d, [lane, w], mi)
                    else:
                        pos = cb * _SCAN + k * _LANES + lane + pos_off
                        packed = (pos << 16) | (pp << 7) | (v & 127)
                        dst = plsc.load_gather(cursor2d, [lane, w])
                        plsc.store_scatter(hits2, [dst], packed, mask=m)
                        plsc.addupdate_scatter(cursor2d, [lane, w], mi)
                    return 0

                lax.fori_loop(0, _SCAN // _LANES, vreg, 0)
                return 0

            lax.fori_loop(0, BATCH // _SCAN, chunk, 0)

        scan_array(tgt_hbm, 0, True)
        scan_array(ctx_hbm, BATCH, True)

        carry = jnp.int32(0)
        for cc in range(8):
            sl = pl.ds(cc * _LANES, _LANES)
            acc = jnp.zeros((_LANES,), jnp.int32)
            for l in range(_LANES):
                acc = acc + counts2d[l, sl]
            cs = jnp.cumsum(acc)
            ex = cs - acc + carry
            offw2d[cc, pl.ds(0, _LANES)] = ex
            ends2d[cc, pl.ds(0, _LANES)] = ex + acc
            carry = carry + cs[_LANES - 1]
            run = ex
            for l in range(_LANES):
                cursor2d[l, sl] = run
                run = run + counts2d[l, sl]

        scan_array(tgt_hbm, 0, False)
        scan_array(ctx_hbm, BATCH, False)

        def win_range(w):
            sw = plsc.load_gather(
                offw2d, [lax.shift_right_logical(w, 4) * jnp.ones(
                    (_LANES,), jnp.int32), jnp.full((_LANES,), w & 15,
                                                    jnp.int32)])
            ew = plsc.load_gather(
                ends2d, [lax.shift_right_logical(w, 4) * jnp.ones(
                    (_LANES,), jnp.int32), jnp.full((_LANES,), w & 15,
                                                    jnp.int32)])
            return sw[0], ew[0]

        def panel_copy(q, sem):
            off = pl.multiple_of((lo + q) * 128, 128)
            slot = q % _RING
            return pltpu.make_async_copy(
                table_hbm.at[:, pl.ds(off, 128)],
                ring.at[pl.ds(slot * EMBED_DIM, EMBED_DIM)],
                sem)

        def flush(scnt):
            def fix(k, _):
                sp = stagepos[pl.ds(k * _LANES, _LANES)]
                rid = k * _LANES + lane
                stagepos[pl.ds(k * _LANES, _LANES)] = jnp.where(
                    rid < scnt, sp, _TRASH + rid)
                return 0

            lax.fori_loop(0, _STAGE // _LANES, fix, 0)
            pltpu.sync_copy(stage, embeds_hbm.at[stagepos])

        def emit(hj, src_ref, rowbase, scnt):
            pos = lax.shift_right_logical(hj, 16)
            cidx = jnp.full((_LANES,), hj & 127, jnp.int32)
            for q in range(EMBED_DIM // _LANES):
                ridx = rowbase + q * _LANES + lane
                vals = plsc.load_gather(src_ref, [ridx, cidx])
                stage[scnt, pl.ds(q * _LANES, _LANES)] = vals
            plsc.store_scatter(stagepos, [jnp.full((_LANES,), scnt, jnp.int32)],
                               jnp.full((_LANES,), pos, jnp.int32),
                               mask=lane == 0)
            return scnt + 1

        def maybe_flush(scnt):
            def do(s):
                flush(s)
                return jnp.int32(0)

            return lax.cond(scnt >= _STAGE - _LANES, do, lambda s: s, scnt)

        def emit_range(w, slot_fn, src_ref, scnt0):
            start, end = win_range(w)

            def vreg(k, scnt):
                scnt = maybe_flush(scnt)
                h = hits2[pl.ds(k * _LANES, _LANES)]
                pp = lax.shift_right_logical(h, 7) & 255
                gi = k * _LANES + lane
                m = ((gi >= start) & (gi < end)).astype(jnp.int32)

                for j in range(_LANES):
                    def do(s, h=h, pp=pp, j=j):
                        return emit(h[j], src_ref, slot_fn(pp[j]) * EMBED_DIM, s)

                    scnt = lax.cond(m[j] != 0, do, lambda s: s, scnt)
                return scnt

            return lax.fori_loop(lax.shift_right_logical(start, 4),
                                 (end + _LANES - 1) // _LANES, vreg, scnt0)

        def process(win, scnt0):
            return emit_range(win, lambda ppj: ppj % _RING, ring, scnt0)

        def special(w, src_ref, scnt0):
            return emit_range(w, lambda ppj: 0, src_ref, scnt0)

        for q in range(_RING):
            panel_copy(jnp.int32(q), sem0).start()

        def window(win, scnt):
            panel_copy(2 * win, sem0).wait()
            panel_copy(2 * win + 1, sem0).wait()
            scnt = process(win, scnt)

            @pl.when(2 * win + _RING < _PPT)
            def _():
                panel_copy(2 * win + _RING, sem0).start()

            @pl.when(2 * win + 1 + _RING < _PPT)
            def _():
                panel_copy(2 * win + 1 + _RING, sem0).start()

            return scnt

        scnt = lax.fori_loop(0, _NWIN, window, jnp.int32(0))

        @pl.when(has_extra)
        def _():
            xoff = pl.multiple_of(xpanel * 128, 128)
            pltpu.make_async_copy(
                table_hbm.at[:, pl.ds(xoff, 128)],
                ring.at[pl.ds(0, EMBED_DIM)], sem1).start()
            pltpu.make_async_copy(
                table_hbm.at[:, pl.ds(xoff, 128)],
                ring.at[pl.ds(0, EMBED_DIM)], sem1).wait()

        scnt = special(jnp.int32(_PP_EXTRA >> 1), ring, scnt)
        scnt = special(jnp.int32(_PP_TAIL >> 1), tailbuf, scnt)
        flush(scnt)

    return k1(target, context, table_t, tail_t)


def _k2_dot(embeds):
    mesh = plsc.VectorSubcoreMesh(core_axis_name="c", subcore_axis_name="s")
    chunk = 128
    nchunk = _BPW // chunk

    @functools.partial(
        pl.kernel,
        mesh=mesh,
        out_type=jax.ShapeDtypeStruct((BATCH,), jnp.float32),
        compiler_params=pltpu.CompilerParams(needs_layout_passes=False),
        scratch_types=[
            pltpu.VMEM((2, chunk, 128), jnp.float32),
            pltpu.VMEM((2, chunk, 128), jnp.float32),
            pltpu.VMEM((_BPW,), jnp.float32),
            pltpu.SemaphoreType.DMA,
            pltpu.SemaphoreType.DMA,
            pltpu.SemaphoreType.DMA,
            pltpu.SemaphoreType.DMA,
        ],
    )
    def k2(embeds_hbm, out_hbm, rows_t, rows_c, scores,
           sem_t0, sem_t1, sem_c0, sem_c1):
        wid = lax.axis_index("s") * _NC + lax.axis_index("c")
        base = wid * _BPW
        sems_t = (sem_t0, sem_t1)
        sems_c = (sem_c0, sem_c1)

        def start(g, slot):
            cpt = pltpu.async_copy(
                embeds_hbm.at[pl.ds(base + g * chunk, chunk)],
                rows_t.at[slot], sems_t[slot])
            cpc = pltpu.async_copy(
                embeds_hbm.at[pl.ds(BATCH + base + g * chunk, chunk)],
                rows_c.at[slot], sems_c[slot])
            return cpt, cpc

        lane = lax.iota(jnp.int32, _LANES)
        inflight = {0: start(0, 0)}

        for g in range(nchunk):
            slot = g % 2
            if g + 1 < nchunk:
                inflight[g + 1] = start(g + 1, (g + 1) % 2)
            cpt, cpc = inflight.pop(g)
            cpt.wait()
            cpc.wait()

            def chunk_body(i, _, slot=slot, g=g):
                vec = jnp.zeros((_LANES,), jnp.float32)
                for j in range(_LANES):
                    r = i * _LANES + j
                    acc = jnp.zeros((_LANES,), jnp.float32)
                    for q in range(EMBED_DIM // _LANES):
                        t = rows_t[slot, r, pl.ds(q * _LANES, _LANES)]
                        c = rows_c[slot, r, pl.ds(q * _LANES, _LANES)]
                        acc = acc + t * c
                    vec = jnp.where(lane == j, jnp.sum(acc), vec)
                scores[pl.ds(g * chunk + i * _LANES, _LANES)] = vec
                return 0

            lax.fori_loop(0, chunk // _LANES, chunk_body, 0)

        pltpu.sync_copy(scores, out_hbm.at[pl.ds(base, _BPW)])

    return k2(embeds)


def kernel(target, context, emb_weight):
    table_t = emb_weight.T
    tail_t = lax.slice(table_t, (0, _TAIL_BASE), (EMBED_DIM, VOCAB))
    embeds = _k1_extract(target.astype(jnp.int32), context.astype(jnp.int32),
                         table_t, tail_t)
    return _k2_dot(embeds)


# speedup vs baseline: 1.2860x; 1.2860x over previous
import functools

import jax
import jax.numpy as jnp
from jax import lax
from jax.experimental import pallas as pl
from jax.experimental.pallas import tpu as pltpu
from jax.experimental.pallas import tpu_sc as plsc

VOCAB = 1000000
EMBED_DIM = 64
BATCH = 16384

_NC = 2
_NS = 16
_NW = _NC * _NS
_BPW = BATCH // _NW
_LANES = 16

_TAIL_BASE = (VOCAB // 128) * 128
_NPANEL = _TAIL_BASE // 128
_PPT = 244
_XBASE = _PPT * _NW
_WIN = 2
_NWIN = _PPT // _WIN
_PP_EXTRA = 252
_PP_TAIL = 254
_NHIT = 32768
_STAGE = 64
_TRASH = 2 * BATCH
_EROWS = 2 * BATCH + _STAGE
_SCAN = 2048


def _k1_extract(target, context, table_t, tail_t):
    mesh = plsc.VectorSubcoreMesh(core_axis_name="c", subcore_axis_name="s")

    @functools.partial(
        pl.kernel,
        mesh=mesh,
        out_type=jax.ShapeDtypeStruct((_EROWS, 128), jnp.float32),
        compiler_params=pltpu.CompilerParams(needs_layout_passes=False),
        scratch_types=[
            pltpu.VMEM((_SCAN,), jnp.int32),
            pltpu.VMEM((_NHIT,), jnp.int32),
            pltpu.VMEM((_NHIT,), jnp.int32),
            pltpu.VMEM((2 * _WIN * EMBED_DIM, 128), jnp.float32),
            pltpu.VMEM((_STAGE, 128), jnp.float32),
            pltpu.VMEM((_STAGE,), jnp.int32),
            pltpu.VMEM((EMBED_DIM, EMBED_DIM), jnp.float32),
            pltpu.VMEM((_LANES,), jnp.int32),
            pltpu.VMEM((_LANES, 128), jnp.int32),
            pltpu.VMEM((_LANES, 128), jnp.int32),
            pltpu.VMEM((8, _LANES), jnp.int32),
            pltpu.VMEM((8, _LANES), jnp.int32),
            pltpu.SemaphoreType.DMA,
            pltpu.SemaphoreType.DMA,
        ],
    )
    def k1(tgt_hbm, ctx_hbm, table_hbm, tail_hbm, embeds_hbm,
           scanbuf, hits, hits2, ring, stage, stagepos, tailbuf,
           lanecur, counts2d, cursor2d, offw2d, ends2d, sem0, sem1):
        wid = lax.axis_index("s") * _NC + lax.axis_index("c")
        lo = wid * _PPT
        xpanel = _XBASE + wid
        has_extra = wid < 4
        is_last = wid == _NW - 1
        lane = lax.iota(jnp.int32, _LANES)
        _SEG = _NHIT // _LANES

        pltpu.sync_copy(tail_hbm, tailbuf)

        lanecur[...] = jnp.zeros((_LANES,), jnp.int32)
        for l in range(_LANES):
            for cc in range(8):
                counts2d[l, pl.ds(cc * _LANES, _LANES)] = jnp.zeros(
                    (_LANES,), jnp.int32)

        def scan_array(arr_hbm, pos_off):
            def chunk(cb, _):
                pltpu.sync_copy(arr_hbm.at[pl.ds(cb * _SCAN, _SCAN)], scanbuf)

                def vreg(k, _):
                    v = scanbuf[pl.ds(k * _LANES, _LANES)]
                    p = lax.shift_right_logical(v, 7)
                    in_main = (p >= lo) & (p < lo + _PPT)
                    is_x = has_extra & (p == xpanel)
                    is_t = is_last & (p == _NPANEL)
                    m = in_main | is_x | is_t
                    pp = jnp.where(is_x, _PP_EXTRA,
                                   jnp.where(is_t, _PP_TAIL, (p - lo) & 255))
                    pos = cb * _SCAN + k * _LANES + lane + pos_off
                    packed = (pos << 16) | (pp << 7) | (v & 127)
                    cur = lanecur[...]
                    plsc.store_scatter(hits, [lane * _SEG + cur], packed,
                                       mask=m)
                    lanecur[...] = cur + m.astype(jnp.int32)
                    return 0

                lax.fori_loop(0, _SCAN // _LANES, vreg, 0)
                return 0

            lax.fori_loop(0, BATCH // _SCAN, chunk, 0)

        scan_array(tgt_hbm, 0)
        scan_array(ctx_hbm, BATCH)
        seglen = lanecur[...]

        for l in range(_LANES):
            len_l = seglen[l]

            def hvreg(k, _, l=l, len_l=len_l):
                h = hits[pl.ds(l * _SEG + k * _LANES, _LANES)]
                valid = (k * _LANES + lane) < len_l
                w = lax.shift_right_logical(
                    lax.shift_right_logical(h, 7) & 255, 1)
                plsc.addupdate_scatter(counts2d, [lane, w],
                                       valid.astype(jnp.int32))
                return 0

            lax.fori_loop(0, (len_l + _LANES - 1) // _LANES, hvreg, 0)

        carry = jnp.int32(0)
        for cc in range(8):
            sl = pl.ds(cc * _LANES, _LANES)
            acc = jnp.zeros((_LANES,), jnp.int32)
            for l in range(_LANES):
                acc = acc + counts2d[l, sl]
            cs = jnp.cumsum(acc)
            ex = cs - acc + carry
            offw2d[cc, pl.ds(0, _LANES)] = ex
            ends2d[cc, pl.ds(0, _LANES)] = ex + acc
            carry = carry + cs[_LANES - 1]
            run = ex
            for l in range(_LANES):
                cursor2d[l, sl] = run
                run = run + counts2d[l, sl]

        for l in range(_LANES):
            len_l = seglen[l]

            def svreg(k, _, l=l, len_l=len_l):
                h = hits[pl.ds(l * _SEG + k * _LANES, _LANES)]
                valid = (k * _LANES + lane) < len_l
                w = lax.shift_right_logical(
                    lax.shift_right_logical(h, 7) & 255, 1)
                dst = plsc.load_gather(cursor2d, [lane, w])
                plsc.store_scatter(hits2, [dst], h, mask=valid)
                plsc.addupdate_scatter(cursor2d, [lane, w],
                                       valid.astype(jnp.int32))
                return 0

            lax.fori_loop(0, (len_l + _LANES - 1) // _LANES, svreg, 0)

        def win_range(w):
            sw = plsc.load_gather(
                offw2d, [lax.shift_right_logical(w, 4) * jnp.ones(
                    (_LANES,), jnp.int32), jnp.full((_LANES,), w & 15,
                                                    jnp.int32)])
            ew = plsc.load_gather(
                ends2d, [lax.shift_right_logical(w, 4) * jnp.ones(
                    (_LANES,), jnp.int32), jnp.full((_LANES,), w & 15,
                                                    jnp.int32)])
            return sw[0], ew[0]

        def panel_copy(p, slot, sem):
            off = pl.multiple_of(p * 128, 128)
            return pltpu.make_async_copy(
                table_hbm.at[:, pl.ds(off, 128)],
                ring.at[pl.ds(slot * EMBED_DIM, EMBED_DIM)],
                sem)

        def fetch_win(win, parity, sem):
            for k in range(_WIN):
                panel_copy(lo + win * _WIN + k, parity * _WIN + k, sem).start()

        def drain_win(win, parity, sem):
            for k in range(_WIN):
                panel_copy(lo + win * _WIN + k, parity * _WIN + k, sem).wait()

        def flush(scnt):
            def fix(k, _):
                sp = stagepos[pl.ds(k * _LANES, _LANES)]
                rid = k * _LANES + lane
                stagepos[pl.ds(k * _LANES, _LANES)] = jnp.where(
                    rid < scnt, sp, _TRASH + rid)
                return 0

            lax.fori_loop(0, _STAGE // _LANES, fix, 0)
            pltpu.sync_copy(stage, embeds_hbm.at[stagepos])

        def emit(hj, src_ref, rowbase, scnt):
            pos = lax.shift_right_logical(hj, 16)
            cidx = jnp.full((_LANES,), hj & 127, jnp.int32)
            for q in range(EMBED_DIM // _LANES):
                ridx = rowbase + q * _LANES + lane
                vals = plsc.load_gather(src_ref, [ridx, cidx])
                stage[scnt, pl.ds(q * _LANES, _LANES)] = vals
            plsc.store_scatter(stagepos, [jnp.full((_LANES,), scnt, jnp.int32)],
                               jnp.full((_LANES,), pos, jnp.int32),
                               mask=lane == 0)
            return scnt + 1

        def maybe_flush(scnt):
            def do(s):
                flush(s)
                return jnp.int32(0)

            return lax.cond(scnt >= _STAGE - _LANES, do, lambda s: s, scnt)

        def emit_range(w, slot_fn, src_ref, scnt0):
            start, end = win_range(w)

            def vreg(k, scnt):
                scnt = maybe_flush(scnt)
                h = hits2[pl.ds(k * _LANES, _LANES)]
                pp = lax.shift_right_logical(h, 7) & 255
                gi = k * _LANES + lane
                m = ((gi >= start) & (gi < end)).astype(jnp.int32)

                for j in range(_LANES):
                    def do(s, h=h, pp=pp, j=j):
                        return emit(h[j], src_ref, slot_fn(pp[j]) * EMBED_DIM, s)

                    scnt = lax.cond(m[j] != 0, do, lambda s: s, scnt)
                return scnt

            return lax.fori_loop(lax.shift_right_logical(start, 4),
                                 (end + _LANES - 1) // _LANES, vreg, scnt0)

        def process(win, parity, scnt0):
            wbase = win * _WIN
            return emit_range(
                win, lambda ppj: parity * _WIN + (ppj - wbase), ring, scnt0)

        def special(w, src_ref, scnt0):
            return emit_range(w, lambda ppj: 0, src_ref, scnt0)

        fetch_win(jnp.int32(0), 0, sem0)

        def pair(t, scnt):
            even = 2 * t
            fetch_win(even + 1, 1, sem1)
            drain_win(even, 0, sem0)
            scnt = process(even, 0, scnt)

            @pl.when(even + 2 < _NWIN)
            def _():
                fetch_win(even + 2, 0, sem0)

            drain_win(even + 1, 1, sem1)
            return process(even + 1, 1, scnt)

        scnt = lax.fori_loop(0, _NWIN // 2, pair, jnp.int32(0))

        @pl.when(has_extra)
        def _():
            panel_copy(xpanel, 0, sem0).start()
            panel_copy(xpanel, 0, sem0).wait()

        scnt = special(jnp.int32(_PP_EXTRA >> 1), ring, scnt)
        scnt = special(jnp.int32(_PP_TAIL >> 1), tailbuf, scnt)
        flush(scnt)

    return k1(target, context, table_t, tail_t)


def _k2_dot(embeds):
    mesh = plsc.VectorSubcoreMesh(core_axis_name="c", subcore_axis_name="s")
    chunk = 128
    nchunk = _BPW // chunk

    @functools.partial(
        pl.kernel,
        mesh=mesh,
        out_type=jax.ShapeDtypeStruct((BATCH,), jnp.float32),
        compiler_params=pltpu.CompilerParams(needs_layout_passes=False),
        scratch_types=[
            pltpu.VMEM((2, chunk, 128), jnp.float32),
            pltpu.VMEM((2, chunk, 128), jnp.float32),
            pltpu.VMEM((_BPW,), jnp.float32),
            pltpu.SemaphoreType.DMA,
            pltpu.SemaphoreType.DMA,
            pltpu.SemaphoreType.DMA,
            pltpu.SemaphoreType.DMA,
        ],
    )
    def k2(embeds_hbm, out_hbm, rows_t, rows_c, scores,
           sem_t0, sem_t1, sem_c0, sem_c1):
        wid = lax.axis_index("s") * _NC + lax.axis_index("c")
        base = wid * _BPW
        sems_t = (sem_t0, sem_t1)
        sems_c = (sem_c0, sem_c1)

        def start(g, slot):
            cpt = pltpu.async_copy(
                embeds_hbm.at[pl.ds(base + g * chunk, chunk)],
                rows_t.at[slot], sems_t[slot])
            cpc = pltpu.async_copy(
                embeds_hbm.at[pl.ds(BATCH + base + g * chunk, chunk)],
                rows_c.at[slot], sems_c[slot])
            return cpt, cpc

        lane = lax.iota(jnp.int32, _LANES)
        inflight = {0: start(0, 0)}

        for g in range(nchunk):
            slot = g % 2
            if g + 1 < nchunk:
                inflight[g + 1] = start(g + 1, (g + 1) % 2)
            cpt, cpc = inflight.pop(g)
            cpt.wait()
            cpc.wait()

            def chunk_body(i, _, slot=slot, g=g):
                vec = jnp.zeros((_LANES,), jnp.float32)
                for j in range(_LANES):
                    r = i * _LANES + j
                    acc = jnp.zeros((_LANES,), jnp.float32)
                    for q in range(EMBED_DIM // _LANES):
                        t = rows_t[slot, r, pl.ds(q * _LANES, _LANES)]
                        c = rows_c[slot, r, pl.ds(q * _LANES, _LANES)]
                        acc = acc + t * c
                    vec = jnp.where(lane == j, jnp.sum(acc), vec)
                scores[pl.ds(g * chunk + i * _LANES, _LANES)] = vec
                return 0

            lax.fori_loop(0, chunk // _LANES, chunk_body, 0)

        pltpu.sync_copy(scores, out_hbm.at[pl.ds(base, _BPW)])

    return k2(embeds)


def kernel(target, context, emb_weight):
    table_t = emb_weight.T
    tail_t = lax.slice(table_t, (0, _TAIL_BASE), (EMBED_DIM, VOCAB))
    embeds = _k1_extract(target.astype(jnp.int32), context.astype(jnp.int32),
                         table_t, tail_t)
    return _k2_dot(embeds)
